# baseline (speedup 1.0000x reference)
import functools
import math

import jax
import jax.numpy as jnp
from jax import lax
from jax.experimental import pallas as pl
from jax.experimental.pallas import tpu as pltpu
from jax.experimental.pallas import tpu_sc as plsc

NUM_BANDS = 32
NORMALIZE_SCALE = 6.87
FDIM = 2 * (1 + 2 * NUM_BANDS)

_TWO_PI = 2.0 * math.pi
_RND = 1.5 * 2.0**23


def _fast_sin(angle):
    n = (angle * (1.0 / _TWO_PI) + _RND) - _RND
    t = angle - n * _TWO_PI
    s = t * t
    p = 2.17325696e-06
    p = p * s + -1.93162699e-04
    p = p * s + 8.31238828e-03
    p = p * s + -1.66632594e-01
    p = p * s + 9.99984593e-01
    return p * t


def _sc_knn(posx, posy, kk):
    B, L = posx.shape
    info = plsc.get_sparse_core_info()
    NC, NS = info.num_cores, info.num_subcores
    NW = NC * NS
    rows_w = (B * L) // NW
    wpb = L // rows_w
    nchunks = L // 16
    mesh = plsc.VectorSubcoreMesh(core_axis_name="c", subcore_axis_name="s")

    @functools.partial(
        pl.kernel,
        mesh=mesh,
        compiler_params=pltpu.CompilerParams(needs_layout_passes=False),
        out_type=[
            jax.ShapeDtypeStruct((NW, rows_w * kk), jnp.int32),
            jax.ShapeDtypeStruct((NW, rows_w * kk), jnp.float32),
            jax.ShapeDtypeStruct((NW, rows_w * kk), jnp.float32),
        ],
        scratch_types=[
            pltpu.VMEM((L,), jnp.float32),
            pltpu.VMEM((L,), jnp.float32),
            pltpu.VMEM((rows_w * kk,), jnp.int32),
            pltpu.VMEM((rows_w * kk,), jnp.float32),
            pltpu.VMEM((rows_w * kk,), jnp.float32),
        ],
    )
    def knn(posx_hbm, posy_hbm, idx_hbm, dx_hbm, dy_hbm, px, py, ib, xb, yb):
        wid = lax.axis_index("s") * NC + lax.axis_index("c")
        batch = wid // wpb
        base = (wid % wpb) * rows_w
        pltpu.sync_copy(posx_hbm.at[batch], px)
        pltpu.sync_copy(posy_hbm.at[batch], py)
        lane = lax.broadcasted_iota(jnp.int32, (16,), 0)
        inf = jnp.float32(jnp.inf)

        def row_body(r, carry):
            q = base + r
            qv = jnp.full((16,), q, jnp.int32)
            xq = plsc.load_gather(px, [qv])
            yq = plsc.load_gather(py, [qv])
            plsc.store_scatter(px, [qv], jnp.full((16,), inf))

            def chunk_body(c, st):
                bd, bi, thr = st
                off = pl.multiple_of(c * 16, 16)
                xj = px[pl.ds(off, 16)]
                yj = py[pl.ds(off, 16)]
                dx = xj - xq
                dy = yj - yq
                d = dx * dx + dy * dy
                cnt = plsc.all_reduce_population_count(d < thr)

                def merge(st2):
                    bd0, bi0, _ = st2
                    ci = c * 16 + lane
                    dd, di = plsc.sort_key_val(d, ci, descending=True)
                    take = dd < bd0
                    nd = jnp.where(take, dd, bd0)
                    ni = jnp.where(take, di, bi0)
                    bd1, bi1 = plsc.sort_key_val(nd, ni)
                    return bd1, bi1, jnp.full((16,), bd1[15])

                return lax.cond(cnt[0] > 0, merge, lambda s: s, (bd, bi, thr))

            init = (jnp.full((16,), inf), jnp.full((16,), L, jnp.int32),
                    jnp.full((16,), inf))
            bd, bi, _ = lax.fori_loop(0, nchunks, chunk_body, init)
            plsc.store_scatter(px, [qv], xq)
            nx = plsc.load_gather(px, [bi])
            ny = plsc.load_gather(py, [bi])
            o = pl.multiple_of(r * kk, kk)
            ib[pl.ds(o, kk)] = bi
            xb[pl.ds(o, kk)] = nx - xq
            yb[pl.ds(o, kk)] = ny - yq
            return carry

        lax.fori_loop(0, rows_w, row_body, 0)
        pltpu.sync_copy(ib, idx_hbm.at[wid])
        pltpu.sync_copy(xb, dx_hbm.at[wid])
        pltpu.sync_copy(yb, dy_hbm.at[wid])

    return knn(posx, posy)


def _encode_body(dx_ref, dy_ref, rpe_ref, dist_ref, self_ref, *, rb, kk):
    dx = dx_ref[...]
    dy = dy_ref[...]
    dist_ref[...] = jnp.sqrt(dx * dx + dy * dy + 1e-8)
    w = kk * FDIM
    p = jax.lax.broadcasted_iota(jnp.int32, (1, w), 1)
    n = p // FDIM
    f = p - n * FDIM
    g = f % 65
    isy = f >= 65
    iscos = g >= 33
    israw = g == 0
    src = n + jnp.where(isy, kk, 0)
    freq = jnp.where(iscos, g - 32, g).astype(jnp.float32)
    phase = jnp.where(iscos, 0.5 * math.pi, 0.0)
    s = jax.lax.broadcasted_iota(jnp.int32, (2 * kk, 1), 0)
    onehot = (s == src).astype(jnp.bfloat16)
    dxy = jnp.concatenate([dx, dy], axis=1)
    h1 = dxy.astype(jnp.bfloat16)
    r1 = dxy - h1.astype(jnp.float32)
    h2 = r1.astype(jnp.bfloat16)
    h3 = (r1 - h2.astype(jnp.float32)).astype(jnp.bfloat16)
    v = 0.0
    for h in (h1, h2, h3):
        v = v + jax.lax.dot_general(
            h, onehot, (((1,), (0,)), ((), ())),
            preferred_element_type=jnp.float32)
    vc = v * (1.0 / NORMALIZE_SCALE)
    vc = vc / (1.0 + jnp.abs(vc))
    enc = _fast_sin(vc * (freq * math.pi) + phase)
    rpe_ref[...] = jnp.where(israw, vc, enc)
    col = jax.lax.broadcasted_iota(jnp.int32, (rb, FDIM), 1)
    self_ref[...] = jnp.where((col % 65) >= 33, 1.0, 0.0)


def kernel(positions, k):
    B, L, _ = positions.shape
    kk = min(16, L - 1)
    posx = positions[..., 0]
    posy = positions[..., 1]

    idx, dxs, dys = _sc_knn(posx, posy, kk)
    idx = idx.reshape(B, L, kk)

    NR = B * L
    RB2 = 64
    grid2 = (NR // RB2,)
    v_spec = pl.BlockSpec((RB2, kk), lambda i: (i, 0))
    rpe, dist, self_rpe = pl.pallas_call(
        functools.partial(_encode_body, rb=RB2, kk=kk),
        grid=grid2,
        in_specs=[v_spec, v_spec],
        out_specs=[
            pl.BlockSpec((RB2, kk * FDIM), lambda i: (i, 0)),
            v_spec,
            pl.BlockSpec((RB2, FDIM), lambda i: (i, 0)),
        ],
        out_shape=[
            jax.ShapeDtypeStruct((NR, kk * FDIM), jnp.float32),
            jax.ShapeDtypeStruct((NR, kk), jnp.float32),
            jax.ShapeDtypeStruct((NR, FDIM), jnp.float32),
        ],
    )(dxs.reshape(NR, kk), dys.reshape(NR, kk))

    topk_indices = idx + jnp.asarray(k - kk, dtype=idx.dtype)
    return (
        topk_indices,
        rpe.reshape(B, L, kk, FDIM),
        self_rpe.reshape(B, L, 1, FDIM),
        dist.reshape(B, L, kk),
    )


# speedup vs baseline: 1.4704x; 1.4704x over previous
import functools
import math

import jax
import jax.numpy as jnp
from jax import lax
from jax.experimental import pallas as pl
from jax.experimental.pallas import tpu as pltpu
from jax.experimental.pallas import tpu_sc as plsc

NUM_BANDS = 32
NORMALIZE_SCALE = 6.87
FDIM = 2 * (1 + 2 * NUM_BANDS)

_TWO_PI = 2.0 * math.pi
_RND = 1.5 * 2.0**23


def _fast_sin(angle):
    n = (angle * (1.0 / _TWO_PI) + _RND) - _RND
    t = angle - n * _TWO_PI
    s = t * t
    p = 2.17325696e-06
    p = p * s + -1.93162699e-04
    p = p * s + 8.31238828e-03
    p = p * s + -1.66632594e-01
    p = p * s + 9.99984593e-01
    return p * t


def _sc_knn(pos_il, L, kk):
    B = pos_il.shape[0]
    info = plsc.get_sparse_core_info()
    NC, NS = info.num_cores, info.num_subcores
    NW = NC * NS
    rows_w = (B * L) // NW
    wpb = L // rows_w
    nchunks = L // 16
    mesh = plsc.VectorSubcoreMesh(core_axis_name="c", subcore_axis_name="s")

    @functools.partial(
        pl.kernel,
        mesh=mesh,
        compiler_params=pltpu.CompilerParams(needs_layout_passes=False),
        out_type=[
            jax.ShapeDtypeStruct((NW, rows_w * kk), jnp.int32),
            jax.ShapeDtypeStruct((NW, rows_w * kk), jnp.float32),
            jax.ShapeDtypeStruct((NW, rows_w * kk), jnp.float32),
        ],
        scratch_types=[
            pltpu.VMEM((2 * L,), jnp.float32),
            pltpu.VMEM((L,), jnp.float32),
            pltpu.VMEM((L + 16,), jnp.float32),
            pltpu.VMEM((L + 16,), jnp.int32),
            pltpu.VMEM((rows_w * kk,), jnp.int32),
            pltpu.VMEM((rows_w * kk,), jnp.float32),
            pltpu.VMEM((rows_w * kk,), jnp.float32),
        ],
    )
    def knn(pos_hbm, idx_hbm, dx_hbm, dy_hbm, pil, db, pd, pi, ib, xb, yb):
        wid = lax.axis_index("s") * NC + lax.axis_index("c")
        batch = wid // wpb
        base = (wid % wpb) * rows_w
        pltpu.sync_copy(pos_hbm.at[batch], pil)
        lane = lax.broadcasted_iota(jnp.int32, (16,), 0)
        lane2 = lane * 2
        inf = jnp.float32(jnp.inf)

        def row_body(r, carry):
            q = base + r
            q2 = jnp.full((16,), 2 * q, jnp.int32)
            xq = plsc.load_gather(pil, [q2])
            yq = plsc.load_gather(pil, [q2 + 1])
            plsc.store_scatter(pil, [q2], jnp.full((16,), inf))

            def a_body(c, m):
                gi2 = c * 32 + lane2
                xj = plsc.load_gather(pil, [gi2])
                yj = plsc.load_gather(pil, [gi2 + 1])
                dx = xj - xq
                dy = yj - yq
                d = dx * dx + dy * dy
                db[pl.ds(pl.multiple_of(c * 16, 16), 16)] = d
                return jnp.minimum(m, d)

            m = lax.fori_loop(0, nchunks, a_body, jnp.full((16,), inf),
                              unroll=4)
            thr0 = jnp.full((16,), jnp.max(m))

            def b_body(c, cnt):
                d = db[pl.ds(pl.multiple_of(c * 16, 16), 16)]
                hit = d <= thr0
                pc = plsc.all_reduce_population_count(hit)
                ci = c * 16 + lane
                plsc.store_compressed(pd.at[pl.ds(cnt, 16)], d, mask=hit)
                plsc.store_compressed(pi.at[pl.ds(cnt, 16)], ci, mask=hit)
                return cnt + pc[0]

            cnt = lax.fori_loop(0, nchunks, b_body, 0, unroll=2)
            pd[pl.ds(cnt, 16)] = jnp.full((16,), inf)
            pi[pl.ds(cnt, 16)] = jnp.full((16,), L, jnp.int32)

            def c_body(c, st):
                bd, bi, thr = st
                d = pd[pl.ds(c * 16, 16)]
                ci = pi[pl.ds(c * 16, 16)]
                hc = plsc.all_reduce_population_count(d < thr)

                def merge(st2):
                    bd0, bi0, _ = st2
                    dd, di = plsc.sort_key_val(d, ci, descending=True)
                    take = dd < bd0
                    nd = jnp.where(take, dd, bd0)
                    ni = jnp.where(take, di, bi0)
                    bd1, bi1 = plsc.sort_key_val(nd, ni)
                    return bd1, bi1, jnp.full((16,), bd1[15])

                return lax.cond(hc[0] > 0, merge, lambda s: s, (bd, bi, thr))

            nit = (cnt + 15) // 16
            init = (jnp.full((16,), inf), jnp.full((16,), L, jnp.int32),
                    jnp.full((16,), inf))
            bd, bi, _ = lax.fori_loop(0, nit, c_body, init)
            plsc.store_scatter(pil, [q2], xq)
            nx = plsc.load_gather(pil, [bi * 2])
            ny = plsc.load_gather(pil, [bi * 2 + 1])
            o = pl.multiple_of(r * kk, kk)
            ib[pl.ds(o, kk)] = bi
            xb[pl.ds(o, kk)] = nx - xq
            yb[pl.ds(o, kk)] = ny - yq
            return carry

        lax.fori_loop(0, rows_w, row_body, 0)
        pltpu.sync_copy(ib, idx_hbm.at[wid])
        pltpu.sync_copy(xb, dx_hbm.at[wid])
        pltpu.sync_copy(yb, dy_hbm.at[wid])

    return knn(pos_il)


def _encode_body(dx_ref, dy_ref, rpe_ref, dist_ref, self_ref, *, rb, kk):
    dx = dx_ref[...]
    dy = dy_ref[...]
    dist_ref[...] = jnp.sqrt(dx * dx + dy * dy + 1e-8)
    w = kk * FDIM
    p = jax.lax.broadcasted_iota(jnp.int32, (1, w), 1)
    n = p // FDIM
    f = p - n * FDIM
    g = f % 65
    isy = f >= 65
    iscos = g >= 33
    israw = g == 0
    src = n + jnp.where(isy, kk, 0)
    freq = jnp.where(iscos, g - 32, g).astype(jnp.float32)
    phase = jnp.where(iscos, 0.5 * math.pi, 0.0)
    s = jax.lax.broadcasted_iota(jnp.int32, (2 * kk, 1), 0)
    onehot = (s == src).astype(jnp.bfloat16)
    dxy = jnp.concatenate([dx, dy], axis=1)
    h1 = dxy.astype(jnp.bfloat16)
    r1 = dxy - h1.astype(jnp.float32)
    h2 = r1.astype(jnp.bfloat16)
    h3 = (r1 - h2.astype(jnp.float32)).astype(jnp.bfloat16)
    v = 0.0
    for h in (h1, h2, h3):
        v = v + jax.lax.dot_general(
            h, onehot, (((1,), (0,)), ((), ())),
            preferred_element_type=jnp.float32)
    vc = v * (1.0 / NORMALIZE_SCALE)
    vc = vc / (1.0 + jnp.abs(vc))
    enc = _fast_sin(vc * (freq * math.pi) + phase)
    rpe_ref[...] = jnp.where(israw, vc, enc)
    col = jax.lax.broadcasted_iota(jnp.int32, (rb, FDIM), 1)
    self_ref[...] = jnp.where((col % 65) >= 33, 1.0, 0.0)


def kernel(positions, k):
    B, L, _ = positions.shape
    kk = min(16, L - 1)
    idx, dxs, dys = _sc_knn(positions.reshape(B, 2 * L), L, kk)
    idx = idx.reshape(B, L, kk)

    NR = B * L
    RB2 = 64
    grid2 = (NR // RB2,)
    v_spec = pl.BlockSpec((RB2, kk), lambda i: (i, 0))
    rpe, dist, self_rpe = pl.pallas_call(
        functools.partial(_encode_body, rb=RB2, kk=kk),
        grid=grid2,
        in_specs=[v_spec, v_spec],
        out_specs=[
            pl.BlockSpec((RB2, kk * FDIM), lambda i: (i, 0)),
            v_spec,
            pl.BlockSpec((RB2, FDIM), lambda i: (i, 0)),
        ],
        out_shape=[
            jax.ShapeDtypeStruct((NR, kk * FDIM), jnp.float32),
            jax.ShapeDtypeStruct((NR, kk), jnp.float32),
            jax.ShapeDtypeStruct((NR, FDIM), jnp.float32),
        ],
    )(dxs.reshape(NR, kk), dys.reshape(NR, kk))

    topk_indices = idx + jnp.asarray(k - kk, dtype=idx.dtype)
    return (
        topk_indices,
        rpe.reshape(B, L, kk, FDIM),
        self_rpe.reshape(B, L, 1, FDIM),
        dist.reshape(B, L, kk),
    )


# speedup vs baseline: 1.5500x; 1.0542x over previous
import functools
import math

import jax
import jax.numpy as jnp
from jax import lax
from jax.experimental import pallas as pl
from jax.experimental.pallas import tpu as pltpu
from jax.experimental.pallas import tpu_sc as plsc

NUM_BANDS = 32
NORMALIZE_SCALE = 6.87
FDIM = 2 * (1 + 2 * NUM_BANDS)

_TWO_PI = 2.0 * math.pi
_RND = 1.5 * 2.0**23


def _fast_sin(angle):
    n = (angle * (1.0 / _TWO_PI) + _RND) - _RND
    t = angle - n * _TWO_PI
    s = t * t
    p = 2.17325696e-06
    p = p * s + -1.93162699e-04
    p = p * s + 8.31238828e-03
    p = p * s + -1.66632594e-01
    p = p * s + 9.99984593e-01
    return p * t


def _sc_knn(pos_il, L, kk):
    B = pos_il.shape[0]
    info = plsc.get_sparse_core_info()
    NC, NS = info.num_cores, info.num_subcores
    NW = NC * NS
    rows_w = (B * L) // NW
    wpb = L // rows_w
    nchunks = L // 16
    mesh = plsc.VectorSubcoreMesh(core_axis_name="c", subcore_axis_name="s")

    @functools.partial(
        pl.kernel,
        mesh=mesh,
        compiler_params=pltpu.CompilerParams(needs_layout_passes=False),
        out_type=[
            jax.ShapeDtypeStruct((NW, rows_w * kk), jnp.int32),
            jax.ShapeDtypeStruct((NW, rows_w * kk), jnp.float32),
            jax.ShapeDtypeStruct((NW, rows_w * kk), jnp.float32),
        ],
        scratch_types=[
            pltpu.VMEM((2 * L,), jnp.float32),
            pltpu.VMEM((L + 16,), jnp.float32),
            pltpu.VMEM((L + 16,), jnp.int32),
            pltpu.VMEM((rows_w * kk,), jnp.int32),
            pltpu.VMEM((rows_w * kk,), jnp.float32),
            pltpu.VMEM((rows_w * kk,), jnp.float32),
        ],
    )
    def knn(pos_hbm, idx_hbm, dx_hbm, dy_hbm, pil, db, pi, ib, xb, yb):
        wid = lax.axis_index("s") * NC + lax.axis_index("c")
        batch = wid // wpb
        base = (wid % wpb) * rows_w
        pltpu.sync_copy(pos_hbm.at[batch], pil)
        db[pl.ds(L, 16)] = jnp.full((16,), jnp.inf)
        lane = lax.broadcasted_iota(jnp.int32, (16,), 0)
        lane2 = lane * 2
        inf = jnp.float32(jnp.inf)

        def row_body(r, carry):
            q = base + r
            q2 = jnp.full((16,), 2 * q, jnp.int32)
            xq = plsc.load_gather(pil, [q2])
            yq = plsc.load_gather(pil, [q2 + 1])
            plsc.store_scatter(pil, [q2], jnp.full((16,), inf))

            def a_body(c, m):
                gi2 = c * 32 + lane2
                xj = plsc.load_gather(pil, [gi2])
                yj = plsc.load_gather(pil, [gi2 + 1])
                dx = xj - xq
                dy = yj - yq
                d = dx * dx + dy * dy
                db[pl.ds(pl.multiple_of(c * 16, 16), 16)] = d
                return jnp.minimum(m, d)

            m = lax.fori_loop(0, nchunks, a_body, jnp.full((16,), inf),
                              unroll=4)
            thr0 = jnp.full((16,), jnp.max(m))

            def b_body(c, cnt):
                d = db[pl.ds(pl.multiple_of(c * 16, 16), 16)]
                hit = d <= thr0
                pc = plsc.all_reduce_population_count(hit)
                ci = c * 16 + lane
                plsc.store_compressed(pi.at[pl.ds(cnt, 16)], ci, mask=hit)
                return cnt + pc[0]

            cnt = lax.fori_loop(0, nchunks, b_body, 0, unroll=2)
            pi[pl.ds(cnt, 16)] = jnp.full((16,), L, jnp.int32)

            def c_body(c, st):
                bd, bi, thr = st
                ci = pi[pl.ds(c * 16, 16)]
                d = plsc.load_gather(db, [ci])
                hc = plsc.all_reduce_population_count(d < thr)

                def merge(st2):
                    bd0, bi0, _ = st2
                    dd, di = plsc.sort_key_val(d, ci, descending=True)
                    take = dd < bd0
                    nd = jnp.where(take, dd, bd0)
                    ni = jnp.where(take, di, bi0)
                    bd1, bi1 = plsc.sort_key_val(nd, ni)
                    return bd1, bi1, jnp.full((16,), bd1[15])

                return lax.cond(hc[0] > 0, merge, lambda s: s, (bd, bi, thr))

            nit = (cnt + 15) // 16
            init = (jnp.full((16,), inf), jnp.full((16,), L, jnp.int32),
                    jnp.full((16,), inf))
            bd, bi, _ = lax.fori_loop(0, nit, c_body, init)
            plsc.store_scatter(pil, [q2], xq)
            nx = plsc.load_gather(pil, [bi * 2])
            ny = plsc.load_gather(pil, [bi * 2 + 1])
            o = pl.multiple_of(r * kk, kk)
            ib[pl.ds(o, kk)] = bi
            xb[pl.ds(o, kk)] = nx - xq
            yb[pl.ds(o, kk)] = ny - yq
            return carry

        lax.fori_loop(0, rows_w, row_body, 0)
        pltpu.sync_copy(ib, idx_hbm.at[wid])
        pltpu.sync_copy(xb, dx_hbm.at[wid])
        pltpu.sync_copy(yb, dy_hbm.at[wid])

    return knn(pos_il)


def _encode_body(dx_ref, dy_ref, rpe_ref, dist_ref, self_ref, *, rb, kk):
    dx = dx_ref[...]
    dy = dy_ref[...]
    dist_ref[...] = jnp.sqrt(dx * dx + dy * dy + 1e-8)
    w = kk * FDIM
    p = jax.lax.broadcasted_iota(jnp.int32, (1, w), 1)
    n = p // FDIM
    f = p - n * FDIM
    g = f % 65
    isy = f >= 65
    iscos = g >= 33
    israw = g == 0
    src = n + jnp.where(isy, kk, 0)
    freq = jnp.where(iscos, g - 32, g).astype(jnp.float32)
    phase = jnp.where(iscos, 0.5 * math.pi, 0.0)
    s = jax.lax.broadcasted_iota(jnp.int32, (2 * kk, 1), 0)
    onehot = (s == src).astype(jnp.bfloat16)
    dxy = jnp.concatenate([dx, dy], axis=1)
    h1 = dxy.astype(jnp.bfloat16)
    r1 = dxy - h1.astype(jnp.float32)
    h2 = r1.astype(jnp.bfloat16)
    h3 = (r1 - h2.astype(jnp.float32)).astype(jnp.bfloat16)
    v = 0.0
    for h in (h1, h2, h3):
        v = v + jax.lax.dot_general(
            h, onehot, (((1,), (0,)), ((), ())),
            preferred_element_type=jnp.float32)
    vc = v * (1.0 / NORMALIZE_SCALE)
    vc = vc / (1.0 + jnp.abs(vc))
    enc = _fast_sin(vc * (freq * math.pi) + phase)
    rpe_ref[...] = jnp.where(israw, vc, enc)
    col = jax.lax.broadcasted_iota(jnp.int32, (rb, FDIM), 1)
    self_ref[...] = jnp.where((col % 65) >= 33, 1.0, 0.0)


def kernel(positions, k):
    B, L, _ = positions.shape
    kk = min(16, L - 1)
    idx, dxs, dys = _sc_knn(positions.reshape(B, 2 * L), L, kk)
    idx = idx.reshape(B, L, kk)

    NR = B * L
    RB2 = 256
    grid2 = (NR // RB2,)
    v_spec = pl.BlockSpec((RB2, kk), lambda i: (i, 0))
    rpe, dist, self_rpe = pl.pallas_call(
        functools.partial(_encode_body, rb=RB2, kk=kk),
        grid=grid2,
        in_specs=[v_spec, v_spec],
        out_specs=[
            pl.BlockSpec((RB2, kk * FDIM), lambda i: (i, 0)),
            v_spec,
            pl.BlockSpec((RB2, FDIM), lambda i: (i, 0)),
        ],
        out_shape=[
            jax.ShapeDtypeStruct((NR, kk * FDIM), jnp.float32),
            jax.ShapeDtypeStruct((NR, kk), jnp.float32),
            jax.ShapeDtypeStruct((NR, FDIM), jnp.float32),
        ],
    )(dxs.reshape(NR, kk), dys.reshape(NR, kk))

    topk_indices = idx + jnp.asarray(k - kk, dtype=idx.dtype)
    return (
        topk_indices,
        rpe.reshape(B, L, kk, FDIM),
        self_rpe.reshape(B, L, 1, FDIM),
        dist.reshape(B, L, kk),
    )


# speedup vs baseline: 2.2226x; 1.4339x over previous
import functools
import math

import jax
import jax.numpy as jnp
from jax import lax
from jax.experimental import pallas as pl
from jax.experimental.pallas import tpu as pltpu
from jax.experimental.pallas import tpu_sc as plsc

NUM_BANDS = 32
NORMALIZE_SCALE = 6.87
FDIM = 2 * (1 + 2 * NUM_BANDS)

_TWO_PI = 2.0 * math.pi
_RND = 1.5 * 2.0**23


def _fast_sin(angle):
    n = (angle * (1.0 / _TWO_PI) + _RND) - _RND
    t = angle - n * _TWO_PI
    s = t * t
    p = 2.17325696e-06
    p = p * s + -1.93162699e-04
    p = p * s + 8.31238828e-03
    p = p * s + -1.66632594e-01
    p = p * s + 9.99984593e-01
    return p * t


def _sc_knn(pos_il, L, kk, S):
    B = pos_il.shape[0]
    info = plsc.get_sparse_core_info()
    NC, NS = info.num_cores, info.num_subcores
    NW = NC * NS
    rows_w = (B * (L - S)) // NW
    wpb = (L - S) // rows_w
    nchunks = L // 16
    mesh = plsc.VectorSubcoreMesh(core_axis_name="c", subcore_axis_name="s")

    @functools.partial(
        pl.kernel,
        mesh=mesh,
        compiler_params=pltpu.CompilerParams(needs_layout_passes=False),
        out_type=[
            jax.ShapeDtypeStruct((NW, rows_w * kk), jnp.int32),
            jax.ShapeDtypeStruct((NW, rows_w * kk), jnp.float32),
            jax.ShapeDtypeStruct((NW, rows_w * kk), jnp.float32),
        ],
        scratch_types=[
            pltpu.VMEM((2 * L,), jnp.float32),
            pltpu.VMEM((L + 16,), jnp.float32),
            pltpu.VMEM((L + 16,), jnp.int32),
            pltpu.VMEM((rows_w * kk,), jnp.int32),
            pltpu.VMEM((rows_w * kk,), jnp.float32),
            pltpu.VMEM((rows_w * kk,), jnp.float32),
        ],
    )
    def knn(pos_hbm, idx_hbm, dx_hbm, dy_hbm, pil, db, pi, ib, xb, yb):
        wid = lax.axis_index("s") * NC + lax.axis_index("c")
        batch = wid // wpb
        base = S + (wid % wpb) * rows_w
        pltpu.sync_copy(pos_hbm.at[batch], pil)
        db[pl.ds(L, 16)] = jnp.full((16,), jnp.inf)
        lane = lax.broadcasted_iota(jnp.int32, (16,), 0)
        lane2 = lane * 2
        inf = jnp.float32(jnp.inf)

        def row_body(r, carry):
            q = base + r
            q2 = jnp.full((16,), 2 * q, jnp.int32)
            xq = plsc.load_gather(pil, [q2])
            yq = plsc.load_gather(pil, [q2 + 1])
            plsc.store_scatter(pil, [q2], jnp.full((16,), inf))

            def a_body(c, m):
                gi2 = c * 32 + lane2
                xj = plsc.load_gather(pil, [gi2])
                yj = plsc.load_gather(pil, [gi2 + 1])
                dx = xj - xq
                dy = yj - yq
                d = dx * dx + dy * dy
                db[pl.ds(pl.multiple_of(c * 16, 16), 16)] = d
                return jnp.minimum(m, d)

            m = lax.fori_loop(0, nchunks, a_body, jnp.full((16,), inf),
                              unroll=4)
            thr0 = jnp.full((16,), jnp.max(m))

            def b_body(c, cnt):
                d = db[pl.ds(pl.multiple_of(c * 16, 16), 16)]
                hit = d <= thr0
                pc = plsc.all_reduce_population_count(hit)
                ci = c * 16 + lane
                plsc.store_compressed(pi.at[pl.ds(cnt, 16)], ci, mask=hit)
                return cnt + pc[0]

            cnt = lax.fori_loop(0, nchunks, b_body, 0, unroll=2)
            pi[pl.ds(cnt, 16)] = jnp.full((16,), L, jnp.int32)

            def c_body(c, st):
                bd, bi, thr = st
                ci = pi[pl.ds(c * 16, 16)]
                d = plsc.load_gather(db, [ci])
                hc = plsc.all_reduce_population_count(d < thr)

                def merge(st2):
                    bd0, bi0, _ = st2
                    dd, di = plsc.sort_key_val(d, ci, descending=True)
                    take = dd < bd0
                    nd = jnp.where(take, dd, bd0)
                    ni = jnp.where(take, di, bi0)
                    bd1, bi1 = plsc.sort_key_val(nd, ni)
                    return bd1, bi1, jnp.full((16,), bd1[15])

                return lax.cond(hc[0] > 0, merge, lambda s: s, (bd, bi, thr))

            nit = (cnt + 15) // 16
            init = (jnp.full((16,), inf), jnp.full((16,), L, jnp.int32),
                    jnp.full((16,), inf))
            bd, bi, _ = lax.fori_loop(0, nit, c_body, init)
            plsc.store_scatter(pil, [q2], xq)
            nx = plsc.load_gather(pil, [bi * 2])
            ny = plsc.load_gather(pil, [bi * 2 + 1])
            o = pl.multiple_of(r * kk, kk)
            ib[pl.ds(o, kk)] = bi
            xb[pl.ds(o, kk)] = nx - xq
            yb[pl.ds(o, kk)] = ny - yq
            return carry

        lax.fori_loop(0, rows_w, row_body, 0)
        pltpu.sync_copy(ib, idx_hbm.at[wid])
        pltpu.sync_copy(xb, dx_hbm.at[wid])
        pltpu.sync_copy(yb, dy_hbm.at[wid])

    return knn(pos_il)


def _topk_body(px_r, py_r, px_c, py_c, idx_ref, dx_ref, dy_ref, *, rb, l, kk):
    xi = px_r[0]
    yi = py_r[0]
    xj = px_c[0]
    yj = py_c[0]
    dxm = xj - xi
    dym = yj - yi
    d = dxm * dxm + dym * dym
    rows = jax.lax.broadcasted_iota(jnp.int32, (rb, l), 0)
    cols = jax.lax.broadcasted_iota(jnp.int32, (rb, l), 1)
    row_base = pl.program_id(1) * rb
    d = jnp.where(cols == rows + row_base, jnp.inf, d)
    for t in range(kk):
        m = jnp.min(d, axis=1, keepdims=True)
        idx_t = jnp.min(jnp.where(d == m, cols, l), axis=1, keepdims=True)
        sel = cols == idx_t
        xj_sel = jnp.sum(jnp.where(sel, dxm, 0.0), axis=1)
        yj_sel = jnp.sum(jnp.where(sel, dym, 0.0), axis=1)
        d = jnp.where(sel, jnp.inf, d)
        idx_ref[0, :, t] = idx_t[:, 0]
        dx_ref[0, :, t] = xj_sel
        dy_ref[0, :, t] = yj_sel


def _encode_body(dx_ref, dy_ref, rpe_ref, dist_ref, self_ref, *, rb, kk):
    dx = dx_ref[...]
    dy = dy_ref[...]
    dist_ref[...] = jnp.sqrt(dx * dx + dy * dy + 1e-8)
    w = kk * FDIM
    p = jax.lax.broadcasted_iota(jnp.int32, (1, w), 1)
    n = p // FDIM
    f = p - n * FDIM
    g = f % 65
    isy = f >= 65
    iscos = g >= 33
    israw = g == 0
    src = n + jnp.where(isy, kk, 0)
    freq = jnp.where(iscos, g - 32, g).astype(jnp.float32)
    phase = jnp.where(iscos, 0.5 * math.pi, 0.0)
    s = jax.lax.broadcasted_iota(jnp.int32, (2 * kk, 1), 0)
    onehot = (s == src).astype(jnp.bfloat16)
    dxy = jnp.concatenate([dx, dy], axis=1)
    h1 = dxy.astype(jnp.bfloat16)
    r1 = dxy - h1.astype(jnp.float32)
    h2 = r1.astype(jnp.bfloat16)
    h3 = (r1 - h2.astype(jnp.float32)).astype(jnp.bfloat16)
    v = 0.0
    for h in (h1, h2, h3):
        v = v + jax.lax.dot_general(
            h, onehot, (((1,), (0,)), ((), ())),
            preferred_element_type=jnp.float32)
    vc = v * (1.0 / NORMALIZE_SCALE)
    vc = vc / (1.0 + jnp.abs(vc))
    enc = _fast_sin(vc * (freq * math.pi) + phase)
    rpe_ref[...] = jnp.where(israw, vc, enc)
    col = jax.lax.broadcasted_iota(jnp.int32, (rb, FDIM), 1)
    self_ref[...] = jnp.where((col % 65) >= 33, 1.0, 0.0)


def kernel(positions, k):
    B, L, _ = positions.shape
    kk = min(16, L - 1)
    S = L // 2
    idx_s, dxs_s, dys_s = _sc_knn(positions.reshape(B, 2 * L), L, kk, S)

    RB = 256
    px_r = positions[..., 0:1]
    py_r = positions[..., 1:2]
    px_c = positions[..., 0].reshape(B, 1, L)
    py_c = positions[..., 1].reshape(B, 1, L)
    grid1 = (B, S // RB)
    r_spec = pl.BlockSpec((1, RB, 1), lambda b, r: (b, r, 0))
    c_spec = pl.BlockSpec((1, 1, L), lambda b, r: (b, 0, 0))
    o_spec = pl.BlockSpec((1, RB, kk), lambda b, r: (b, r, 0))
    idx_t, dxs_t, dys_t = pl.pallas_call(
        functools.partial(_topk_body, rb=RB, l=L, kk=kk),
        grid=grid1,
        in_specs=[r_spec, r_spec, c_spec, c_spec],
        out_specs=[o_spec, o_spec, o_spec],
        out_shape=[
            jax.ShapeDtypeStruct((B, S, kk), jnp.int32),
            jax.ShapeDtypeStruct((B, S, kk), jnp.float32),
            jax.ShapeDtypeStruct((B, S, kk), jnp.float32),
        ],
    )(px_r, py_r, px_c, py_c)

    idx = jnp.concatenate([idx_t, idx_s.reshape(B, L - S, kk)], axis=1)
    dxs = jnp.concatenate([dxs_t, dxs_s.reshape(B, L - S, kk)], axis=1)
    dys = jnp.concatenate([dys_t, dys_s.reshape(B, L - S, kk)], axis=1)

    NR = B * L
    RB2 = 256
    grid2 = (NR // RB2,)
    v_spec = pl.BlockSpec((RB2, kk), lambda i: (i, 0))
    rpe, dist, self_rpe = pl.pallas_call(
        functools.partial(_encode_body, rb=RB2, kk=kk),
        grid=grid2,
        in_specs=[v_spec, v_spec],
        out_specs=[
            pl.BlockSpec((RB2, kk * FDIM), lambda i: (i, 0)),
            v_spec,
            pl.BlockSpec((RB2, FDIM), lambda i: (i, 0)),
        ],
        out_shape=[
            jax.ShapeDtypeStruct((NR, kk * FDIM), jnp.float32),
            jax.ShapeDtypeStruct((NR, kk), jnp.float32),
            jax.ShapeDtypeStruct((NR, FDIM), jnp.float32),
        ],
    )(dxs.reshape(NR, kk), dys.reshape(NR, kk))

    topk_indices = idx + jnp.asarray(k - kk, dtype=idx.dtype)
    return (
        topk_indices,
        rpe.reshape(B, L, kk, FDIM),
        self_rpe.reshape(B, L, 1, FDIM),
        dist.reshape(B, L, kk),
    )


# speedup vs baseline: 2.4425x; 1.0990x over previous
import functools
import math

import jax
import jax.numpy as jnp
from jax import lax
from jax.experimental import pallas as pl
from jax.experimental.pallas import tpu as pltpu
from jax.experimental.pallas import tpu_sc as plsc

NUM_BANDS = 32
NORMALIZE_SCALE = 6.87
FDIM = 2 * (1 + 2 * NUM_BANDS)

_TWO_PI = 2.0 * math.pi
_RND = 1.5 * 2.0**23


def _fast_sin(angle):
    n = (angle * (1.0 / _TWO_PI) + _RND) - _RND
    t = angle - n * _TWO_PI
    s = t * t
    p = 2.17325696e-06
    p = p * s + -1.93162699e-04
    p = p * s + 8.31238828e-03
    p = p * s + -1.66632594e-01
    p = p * s + 9.99984593e-01
    return p * t


def _sc_knn(pos_il, L, kk, S):
    B = pos_il.shape[0]
    info = plsc.get_sparse_core_info()
    NC, NS = info.num_cores, info.num_subcores
    NW = NC * NS
    rows_w = (B * (L - S)) // NW
    wpb = (L - S) // rows_w
    nchunks = L // 16
    mesh = plsc.VectorSubcoreMesh(core_axis_name="c", subcore_axis_name="s")

    @functools.partial(
        pl.kernel,
        mesh=mesh,
        compiler_params=pltpu.CompilerParams(needs_layout_passes=False),
        out_type=[
            jax.ShapeDtypeStruct((NW, rows_w * kk), jnp.int32),
            jax.ShapeDtypeStruct((NW, rows_w * kk), jnp.float32),
            jax.ShapeDtypeStruct((NW, rows_w * kk), jnp.float32),
        ],
        scratch_types=[
            pltpu.VMEM((2 * L,), jnp.float32),
            pltpu.VMEM((L + 16,), jnp.float32),
            pltpu.VMEM((L + 16,), jnp.int32),
            pltpu.VMEM((rows_w * kk,), jnp.int32),
            pltpu.VMEM((rows_w * kk,), jnp.float32),
            pltpu.VMEM((rows_w * kk,), jnp.float32),
        ],
    )
    def knn(pos_hbm, idx_hbm, dx_hbm, dy_hbm, pil, db, pi, ib, xb, yb):
        wid = lax.axis_index("s") * NC + lax.axis_index("c")
        batch = wid // wpb
        base = S + (wid % wpb) * rows_w
        pltpu.sync_copy(pos_hbm.at[batch], pil)
        db[pl.ds(L, 16)] = jnp.full((16,), jnp.inf)
        lane = lax.broadcasted_iota(jnp.int32, (16,), 0)
        lane2 = lane * 2
        inf = jnp.float32(jnp.inf)

        def row_body(r, carry):
            q = base + r
            q2 = jnp.full((16,), 2 * q, jnp.int32)
            xq = plsc.load_gather(pil, [q2])
            yq = plsc.load_gather(pil, [q2 + 1])
            plsc.store_scatter(pil, [q2], jnp.full((16,), inf))

            def a_body(c, m):
                gi2 = c * 32 + lane2
                xj = plsc.load_gather(pil, [gi2])
                yj = plsc.load_gather(pil, [gi2 + 1])
                dx = xj - xq
                dy = yj - yq
                d = dx * dx + dy * dy
                db[pl.ds(pl.multiple_of(c * 16, 16), 16)] = d
                return jnp.minimum(m, d)

            m = lax.fori_loop(0, nchunks, a_body, jnp.full((16,), inf),
                              unroll=4)
            thr0 = jnp.full((16,), jnp.max(m))

            def b_body(c, cnt):
                d = db[pl.ds(pl.multiple_of(c * 16, 16), 16)]
                hit = d <= thr0
                pc = plsc.all_reduce_population_count(hit)
                ci = c * 16 + lane
                plsc.store_compressed(pi.at[pl.ds(cnt, 16)], ci, mask=hit)
                return cnt + pc[0]

            cnt = lax.fori_loop(0, nchunks, b_body, 0, unroll=2)
            pi[pl.ds(cnt, 16)] = jnp.full((16,), L, jnp.int32)

            def c_body(c, st):
                bd, bi, thr = st
                ci = pi[pl.ds(c * 16, 16)]
                d = plsc.load_gather(db, [ci])
                hc = plsc.all_reduce_population_count(d < thr)

                def merge(st2):
                    bd0, bi0, _ = st2
                    dd, di = plsc.sort_key_val(d, ci, descending=True)
                    take = dd < bd0
                    nd = jnp.where(take, dd, bd0)
                    ni = jnp.where(take, di, bi0)
                    bd1, bi1 = plsc.sort_key_val(nd, ni)
                    return bd1, bi1, jnp.full((16,), bd1[15])

                return lax.cond(hc[0] > 0, merge, lambda s: s, (bd, bi, thr))

            nit = (cnt + 15) // 16
            init = (jnp.full((16,), inf), jnp.full((16,), L, jnp.int32),
                    jnp.full((16,), inf))
            bd, bi, _ = lax.fori_loop(0, nit, c_body, init)
            plsc.store_scatter(pil, [q2], xq)
            nx = plsc.load_gather(pil, [bi * 2])
            ny = plsc.load_gather(pil, [bi * 2 + 1])
            o = pl.multiple_of(r * kk, kk)
            ib[pl.ds(o, kk)] = bi
            xb[pl.ds(o, kk)] = nx - xq
            yb[pl.ds(o, kk)] = ny - yq
            return carry

        lax.fori_loop(0, rows_w, row_body, 0)
        pltpu.sync_copy(ib, idx_hbm.at[wid])
        pltpu.sync_copy(xb, dx_hbm.at[wid])
        pltpu.sync_copy(yb, dy_hbm.at[wid])

    return knn(pos_il)


def _topk_body(px_r, py_r, px_c, py_c, idx_ref, dx_ref, dy_ref, *, rb, l, kk):
    xi = px_r[0]
    yi = py_r[0]
    xj = px_c[0]
    yj = py_c[0]
    dxm = xj - xi
    dym = yj - yi
    d = dxm * dxm + dym * dym
    rows = jax.lax.broadcasted_iota(jnp.int32, (rb, l), 0)
    cols = jax.lax.broadcasted_iota(jnp.int32, (rb, l), 1)
    row_base = pl.program_id(1) * rb
    d = jnp.where(cols == rows + row_base, jnp.inf, d)
    for t in range(kk):
        m = jnp.min(d, axis=1, keepdims=True)
        idx_t = jnp.min(jnp.where(d == m, cols, l), axis=1, keepdims=True)
        sel = cols == idx_t
        xj_sel = jnp.sum(jnp.where(sel, dxm, 0.0), axis=1)
        yj_sel = jnp.sum(jnp.where(sel, dym, 0.0), axis=1)
        d = jnp.where(sel, jnp.inf, d)
        idx_ref[0, :, t] = idx_t[:, 0]
        dx_ref[0, :, t] = xj_sel
        dy_ref[0, :, t] = yj_sel


def _encode_body(dx_ref, dy_ref, rpe_ref, dist_ref, self_ref, *, rb, kk):
    dx = dx_ref[...]
    dy = dy_ref[...]
    dist_ref[...] = jnp.sqrt(dx * dx + dy * dy + 1e-8)
    w = kk * FDIM
    p = jax.lax.broadcasted_iota(jnp.int32, (1, w), 1)
    n = p // FDIM
    f = p - n * FDIM
    g = f % 65
    isy = f >= 65
    iscos = g >= 33
    israw = g == 0
    src = n + jnp.where(isy, kk, 0)
    freq = jnp.where(iscos, g - 32, g).astype(jnp.float32)
    phase = jnp.where(iscos, 0.5 * math.pi, 0.0)
    s = jax.lax.broadcasted_iota(jnp.int32, (2 * kk, 1), 0)
    onehot = (s == src).astype(jnp.bfloat16)
    dxy = jnp.concatenate([dx, dy], axis=1)
    h1 = dxy.astype(jnp.bfloat16)
    r1 = dxy - h1.astype(jnp.float32)
    h2 = r1.astype(jnp.bfloat16)
    h3 = (r1 - h2.astype(jnp.float32)).astype(jnp.bfloat16)
    v = 0.0
    for h in (h1, h2, h3):
        v = v + jax.lax.dot_general(
            h, onehot, (((1,), (0,)), ((), ())),
            preferred_element_type=jnp.float32)
    vc = v * (1.0 / NORMALIZE_SCALE)
    vc = vc / (1.0 + jnp.abs(vc))
    enc = _fast_sin(vc * (freq * math.pi) + phase)
    rpe_ref[...] = jnp.where(israw, vc, enc)
    col = jax.lax.broadcasted_iota(jnp.int32, (rb, FDIM), 1)
    self_ref[...] = jnp.where((col % 65) >= 33, 1.0, 0.0)


def kernel(positions, k):
    B, L, _ = positions.shape
    kk = min(16, L - 1)
    S = L // 2
    idx_s, dxs_s, dys_s = _sc_knn(positions.reshape(B, 2 * L), L, kk, S)

    RB = 256
    px_r = positions[..., 0:1]
    py_r = positions[..., 1:2]
    px_c = positions[..., 0].reshape(B, 1, L)
    py_c = positions[..., 1].reshape(B, 1, L)
    grid1 = (B, S // RB)
    r_spec = pl.BlockSpec((1, RB, 1), lambda b, r: (b, r, 0))
    c_spec = pl.BlockSpec((1, 1, L), lambda b, r: (b, 0, 0))
    o_spec = pl.BlockSpec((1, RB, kk), lambda b, r: (b, r, 0))
    idx_t, dxs_t, dys_t = pl.pallas_call(
        functools.partial(_topk_body, rb=RB, l=L, kk=kk),
        grid=grid1,
        in_specs=[r_spec, r_spec, c_spec, c_spec],
        out_specs=[o_spec, o_spec, o_spec],
        out_shape=[
            jax.ShapeDtypeStruct((B, S, kk), jnp.int32),
            jax.ShapeDtypeStruct((B, S, kk), jnp.float32),
            jax.ShapeDtypeStruct((B, S, kk), jnp.float32),
        ],
    )(px_r, py_r, px_c, py_c)

    idx = jnp.concatenate([idx_t, idx_s.reshape(B, L - S, kk)], axis=1)

    def encode(dxh, dyh):
        NRh = dxh.shape[0]
        RB2 = 256
        v_spec = pl.BlockSpec((RB2, kk), lambda i: (i, 0))
        return pl.pallas_call(
            functools.partial(_encode_body, rb=RB2, kk=kk),
            grid=(NRh // RB2,),
            in_specs=[v_spec, v_spec],
            out_specs=[
                pl.BlockSpec((RB2, kk * FDIM), lambda i: (i, 0)),
                v_spec,
                pl.BlockSpec((RB2, FDIM), lambda i: (i, 0)),
            ],
            out_shape=[
                jax.ShapeDtypeStruct((NRh, kk * FDIM), jnp.float32),
                jax.ShapeDtypeStruct((NRh, kk), jnp.float32),
                jax.ShapeDtypeStruct((NRh, FDIM), jnp.float32),
            ],
        )(dxh, dyh)

    rpe_t, dist_t, self_t = encode(dxs_t.reshape(B * S, kk),
                                   dys_t.reshape(B * S, kk))
    rpe_s, dist_s, self_s = encode(dxs_s.reshape(B * (L - S), kk),
                                   dys_s.reshape(B * (L - S), kk))

    rpe = jnp.concatenate([rpe_t.reshape(B, S, kk, FDIM),
                           rpe_s.reshape(B, L - S, kk, FDIM)], axis=1)
    dist = jnp.concatenate([dist_t.reshape(B, S, kk),
                            dist_s.reshape(B, L - S, kk)], axis=1)
    self_rpe = jnp.concatenate([self_t.reshape(B, S, 1, FDIM),
                                self_s.reshape(B, L - S, 1, FDIM)], axis=1)
    topk_indices = idx + jnp.asarray(k - kk, dtype=idx.dtype)
    return (topk_indices, rpe, self_rpe, dist)


# speedup vs baseline: 2.4460x; 1.0015x over previous
import functools
import math

import jax
import jax.numpy as jnp
from jax import lax
from jax.experimental import pallas as pl
from jax.experimental.pallas import tpu as pltpu
from jax.experimental.pallas import tpu_sc as plsc

NUM_BANDS = 32
NORMALIZE_SCALE = 6.87
FDIM = 2 * (1 + 2 * NUM_BANDS)

_TWO_PI = 2.0 * math.pi
_RND = 1.5 * 2.0**23


def _fast_sin(angle):
    n = (angle * (1.0 / _TWO_PI) + _RND) - _RND
    t = angle - n * _TWO_PI
    s = t * t
    p = 2.17325696e-06
    p = p * s + -1.93162699e-04
    p = p * s + 8.31238828e-03
    p = p * s + -1.66632594e-01
    p = p * s + 9.99984593e-01
    return p * t


def _sc_knn(pos_il, L, kk, S):
    B = pos_il.shape[0]
    info = plsc.get_sparse_core_info()
    NC, NS = info.num_cores, info.num_subcores
    NW = NC * NS
    rows_w = (B * (L - S)) // NW
    wpb = (L - S) // rows_w
    nchunks = L // 16
    mesh = plsc.VectorSubcoreMesh(core_axis_name="c", subcore_axis_name="s")

    @functools.partial(
        pl.kernel,
        mesh=mesh,
        compiler_params=pltpu.CompilerParams(needs_layout_passes=False),
        out_type=[
            jax.ShapeDtypeStruct((B * (L - S), kk), jnp.int32),
            jax.ShapeDtypeStruct((B * (L - S), kk), jnp.float32),
            jax.ShapeDtypeStruct((B * (L - S), kk), jnp.float32),
        ],
        scratch_types=[
            pltpu.VMEM((2 * L,), jnp.float32),
            pltpu.VMEM((L + 16,), jnp.float32),
            pltpu.VMEM((L + 16,), jnp.int32),
            pltpu.VMEM((rows_w, kk), jnp.int32),
            pltpu.VMEM((rows_w, kk), jnp.float32),
            pltpu.VMEM((rows_w, kk), jnp.float32),
        ],
    )
    def knn(pos_hbm, idx_hbm, dx_hbm, dy_hbm, pil, db, pi, ib, xb, yb):
        wid = lax.axis_index("s") * NC + lax.axis_index("c")
        batch = wid // wpb
        base = S + (wid % wpb) * rows_w
        pltpu.sync_copy(pos_hbm.at[batch], pil)
        db[pl.ds(L, 16)] = jnp.full((16,), jnp.inf)
        lane = lax.broadcasted_iota(jnp.int32, (16,), 0)
        lane2 = lane * 2
        inf = jnp.float32(jnp.inf)

        def row_body(r, carry):
            q = base + r
            q2 = jnp.full((16,), 2 * q, jnp.int32)
            xq = plsc.load_gather(pil, [q2])
            yq = plsc.load_gather(pil, [q2 + 1])
            plsc.store_scatter(pil, [q2], jnp.full((16,), inf))

            def a_body(c, m):
                gi2 = c * 32 + lane2
                xj = plsc.load_gather(pil, [gi2])
                yj = plsc.load_gather(pil, [gi2 + 1])
                dx = xj - xq
                dy = yj - yq
                d = dx * dx + dy * dy
                db[pl.ds(pl.multiple_of(c * 16, 16), 16)] = d
                return jnp.minimum(m, d)

            m = lax.fori_loop(0, nchunks, a_body, jnp.full((16,), inf),
                              unroll=4)
            thr0 = jnp.full((16,), jnp.max(m))

            def b_body(c, cnt):
                d = db[pl.ds(pl.multiple_of(c * 16, 16), 16)]
                hit = d <= thr0
                pc = plsc.all_reduce_population_count(hit)
                ci = c * 16 + lane
                plsc.store_compressed(pi.at[pl.ds(cnt, 16)], ci, mask=hit)
                return cnt + pc[0]

            cnt = lax.fori_loop(0, nchunks, b_body, 0, unroll=2)
            pi[pl.ds(cnt, 16)] = jnp.full((16,), L, jnp.int32)

            def c_body(c, st):
                bd, bi, thr = st
                ci = pi[pl.ds(c * 16, 16)]
                d = plsc.load_gather(db, [ci])
                hc = plsc.all_reduce_population_count(d < thr)

                def merge(st2):
                    bd0, bi0, _ = st2
                    dd, di = plsc.sort_key_val(d, ci, descending=True)
                    take = dd < bd0
                    nd = jnp.where(take, dd, bd0)
                    ni = jnp.where(take, di, bi0)
                    bd1, bi1 = plsc.sort_key_val(nd, ni)
                    return bd1, bi1, jnp.full((16,), bd1[15])

                return lax.cond(hc[0] > 0, merge, lambda s: s, (bd, bi, thr))

            nit = (cnt + 15) // 16
            init = (jnp.full((16,), inf), jnp.full((16,), L, jnp.int32),
                    jnp.full((16,), inf))
            bd, bi, _ = lax.fori_loop(0, nit, c_body, init)
            plsc.store_scatter(pil, [q2], xq)
            nx = plsc.load_gather(pil, [bi * 2])
            ny = plsc.load_gather(pil, [bi * 2 + 1])
            ib[r, :] = bi
            xb[r, :] = nx - xq
            yb[r, :] = ny - yq
            return carry

        lax.fori_loop(0, rows_w, row_body, 0)
        gbase = batch * (L - S) + (wid % wpb) * rows_w
        pltpu.sync_copy(ib, idx_hbm.at[pl.ds(gbase, rows_w), :])
        pltpu.sync_copy(xb, dx_hbm.at[pl.ds(gbase, rows_w), :])
        pltpu.sync_copy(yb, dy_hbm.at[pl.ds(gbase, rows_w), :])

    return knn(pos_il)


def _topk_body(px_r, py_r, px_c, py_c, idx_ref, dx_ref, dy_ref, *, rb, l, kk):
    xi = px_r[0]
    yi = py_r[0]
    xj = px_c[0]
    yj = py_c[0]
    dxm = xj - xi
    dym = yj - yi
    d = dxm * dxm + dym * dym
    rows = jax.lax.broadcasted_iota(jnp.int32, (rb, l), 0)
    cols = jax.lax.broadcasted_iota(jnp.int32, (rb, l), 1)
    row_base = pl.program_id(1) * rb
    d = jnp.where(cols == rows + row_base, jnp.inf, d)
    for t in range(kk):
        m = jnp.min(d, axis=1, keepdims=True)
        idx_t = jnp.min(jnp.where(d == m, cols, l), axis=1, keepdims=True)
        sel = cols == idx_t
        xj_sel = jnp.sum(jnp.where(sel, dxm, 0.0), axis=1)
        yj_sel = jnp.sum(jnp.where(sel, dym, 0.0), axis=1)
        d = jnp.where(sel, jnp.inf, d)
        idx_ref[0, :, t] = idx_t[:, 0]
        dx_ref[0, :, t] = xj_sel
        dy_ref[0, :, t] = yj_sel


def _encode_body(dx_ref, dy_ref, rpe_ref, dist_ref, self_ref, *, rb, kk):
    dx = dx_ref[...]
    dy = dy_ref[...]
    dist_ref[...] = jnp.sqrt(dx * dx + dy * dy + 1e-8)
    w = kk * FDIM
    p = jax.lax.broadcasted_iota(jnp.int32, (1, w), 1)
    n = p // FDIM
    f = p - n * FDIM
    g = f % 65
    isy = f >= 65
    iscos = g >= 33
    israw = g == 0
    src = n + jnp.where(isy, kk, 0)
    freq = jnp.where(iscos, g - 32, g).astype(jnp.float32)
    phase = jnp.where(iscos, 0.5 * math.pi, 0.0)
    s = jax.lax.broadcasted_iota(jnp.int32, (2 * kk, 1), 0)
    onehot = (s == src).astype(jnp.bfloat16)
    dxy = jnp.concatenate([dx, dy], axis=1)
    h1 = dxy.astype(jnp.bfloat16)
    r1 = dxy - h1.astype(jnp.float32)
    h2 = r1.astype(jnp.bfloat16)
    h3 = (r1 - h2.astype(jnp.float32)).astype(jnp.bfloat16)
    v = 0.0
    for h in (h1, h2, h3):
        v = v + jax.lax.dot_general(
            h, onehot, (((1,), (0,)), ((), ())),
            preferred_element_type=jnp.float32)
    vc = v * (1.0 / NORMALIZE_SCALE)
    vc = vc / (1.0 + jnp.abs(vc))
    enc = _fast_sin(vc * (freq * math.pi) + phase)
    rpe_ref[...] = jnp.where(israw, vc, enc)
    col = jax.lax.broadcasted_iota(jnp.int32, (rb, FDIM), 1)
    self_ref[...] = jnp.where((col % 65) >= 33, 1.0, 0.0)


def kernel(positions, k):
    B, L, _ = positions.shape
    kk = min(16, L - 1)
    S = L // 2
    idx_s, dxs_s, dys_s = _sc_knn(positions.reshape(B, 2 * L), L, kk, S)

    RB = 256
    px_r = positions[..., 0:1]
    py_r = positions[..., 1:2]
    px_c = positions[..., 0].reshape(B, 1, L)
    py_c = positions[..., 1].reshape(B, 1, L)
    grid1 = (B, S // RB)
    r_spec = pl.BlockSpec((1, RB, 1), lambda b, r: (b, r, 0))
    c_spec = pl.BlockSpec((1, 1, L), lambda b, r: (b, 0, 0))
    o_spec = pl.BlockSpec((1, RB, kk), lambda b, r: (b, r, 0))
    idx_t, dxs_t, dys_t = pl.pallas_call(
        functools.partial(_topk_body, rb=RB, l=L, kk=kk),
        grid=grid1,
        in_specs=[r_spec, r_spec, c_spec, c_spec],
        out_specs=[o_spec, o_spec, o_spec],
        out_shape=[
            jax.ShapeDtypeStruct((B, S, kk), jnp.int32),
            jax.ShapeDtypeStruct((B, S, kk), jnp.float32),
            jax.ShapeDtypeStruct((B, S, kk), jnp.float32),
        ],
    )(px_r, py_r, px_c, py_c)

    idx = jnp.concatenate([idx_t, idx_s.reshape(B, L - S, kk)], axis=1)

    def encode(dxh, dyh):
        NRh = dxh.shape[0]
        RB2 = 256
        v_spec = pl.BlockSpec((RB2, kk), lambda i: (i, 0))
        return pl.pallas_call(
            functools.partial(_encode_body, rb=RB2, kk=kk),
            grid=(NRh // RB2,),
            in_specs=[v_spec, v_spec],
            out_specs=[
                pl.BlockSpec((RB2, kk * FDIM), lambda i: (i, 0)),
                v_spec,
                pl.BlockSpec((RB2, FDIM), lambda i: (i, 0)),
            ],
            out_shape=[
                jax.ShapeDtypeStruct((NRh, kk * FDIM), jnp.float32),
                jax.ShapeDtypeStruct((NRh, kk), jnp.float32),
                jax.ShapeDtypeStruct((NRh, FDIM), jnp.float32),
            ],
        )(dxh, dyh)

    rpe_t, dist_t, self_t = encode(dxs_t.reshape(B * S, kk),
                                   dys_t.reshape(B * S, kk))
    rpe_s, dist_s, self_s = encode(dxs_s, dys_s)

    rpe = jnp.concatenate([rpe_t.reshape(B, S, kk, FDIM),
                           rpe_s.reshape(B, L - S, kk, FDIM)], axis=1)
    dist = jnp.concatenate([dist_t.reshape(B, S, kk),
                            dist_s.reshape(B, L - S, kk)], axis=1)
    self_rpe = jnp.concatenate([self_t.reshape(B, S, 1, FDIM),
                                self_s.reshape(B, L - S, 1, FDIM)], axis=1)
    topk_indices = idx + jnp.asarray(k - kk, dtype=idx.dtype)
    return (topk_indices, rpe, self_rpe, dist)
